# baseline (speedup 1.0000x reference)
import functools

import jax
import jax.numpy as jnp
from jax import lax
from jax.experimental import pallas as pl
from jax.experimental.pallas import tpu as pltpu
from jax.experimental.pallas import tpu_sc as plsc

B = 4096
L = 50
V = 1000000
D = 32

N = B * L
NC = 2
NS = 16
NW = NC * NS
PER_W = N // NW
CH = 1600
NCH = PER_W // CH

_mesh = plsc.VectorSubcoreMesh(core_axis_name="c", subcore_axis_name="s")


@functools.partial(
    pl.kernel,
    mesh=_mesh,
    out_type=jax.ShapeDtypeStruct((N, D), jnp.float32),
    compiler_params=pltpu.CompilerParams(use_tc_tiling_on_sc=False),
    scratch_types=[
        pltpu.VMEM((PER_W,), jnp.int32),
        pltpu.VMEM((CH, D), jnp.float32),
        pltpu.VMEM((CH, D), jnp.float32),
        pltpu.SemaphoreType.DMA,
        pltpu.SemaphoreType.DMA,
    ],
)
def _gather_kernel(idx_hbm, table_hbm, out_hbm, idx_v, rows0, rows1, sem0, sem1):
    wid = lax.axis_index("s") * NC + lax.axis_index("c")
    base = wid * PER_W

    pltpu.sync_copy(idx_hbm.at[pl.ds(base, PER_W)], idx_v)

    rows = (rows0, rows1)
    sems = (sem0, sem1)

    copies = [None, None]
    copies[0] = pltpu.async_copy(table_hbm.at[idx_v.at[pl.ds(0, CH)]], rows[0], sems[0])
    for c in range(NCH):
        nxt = (c + 1) % 2
        if c + 1 < NCH:
            copies[nxt] = pltpu.async_copy(
                table_hbm.at[idx_v.at[pl.ds((c + 1) * CH, CH)]], rows[nxt], sems[nxt]
            )
        copies[c % 2].wait()
        pltpu.sync_copy(rows[c % 2], out_hbm.at[pl.ds(base + c * CH, CH)])


def kernel(indices, table):
    flat_idx = indices.reshape(N).astype(jnp.int32)
    out = _gather_kernel(flat_idx, table)
    word_embeddings = out.reshape(B, L, D)
    lengths = jnp.full((B,), L, dtype=jnp.int32)
    return (word_embeddings, lengths)


# speedup vs baseline: 1.0005x; 1.0005x over previous
import functools

import jax
import jax.numpy as jnp
from jax import lax
from jax.experimental import pallas as pl
from jax.experimental.pallas import tpu as pltpu
from jax.experimental.pallas import tpu_sc as plsc

B = 4096
L = 50
V = 1000000
D = 32

N = B * L
NC = 2
NS = 16
NW = NC * NS
PER_W = N // NW
CH = 1600
NCH = PER_W // CH

_mesh = plsc.VectorSubcoreMesh(core_axis_name="c", subcore_axis_name="s")


@functools.partial(
    pl.kernel,
    mesh=_mesh,
    compiler_params=pltpu.CompilerParams(use_tc_tiling_on_sc=False),
    out_type=jax.ShapeDtypeStruct((N, D), jnp.float32),
    scratch_types=[
        pltpu.VMEM((PER_W,), jnp.int32),
        pltpu.VMEM((CH, D), jnp.float32),
        pltpu.VMEM((CH, D), jnp.float32),
        pltpu.SemaphoreType.DMA,
        pltpu.SemaphoreType.DMA,
    ],
)
def _gather_kernel(idx_hbm, table_hbm, out_hbm, idx_v, rows0, rows1, sem0, sem1):
    wid = lax.axis_index("s") * NC + lax.axis_index("c")
    base = wid * PER_W

    pltpu.sync_copy(idx_hbm.at[pl.ds(base, PER_W)], idx_v)

    rows = (rows0, rows1)
    sems = (sem0, sem1)

    copies = [None, None]
    copies[0] = pltpu.async_copy(table_hbm.at[idx_v.at[pl.ds(0, CH)]], rows[0], sems[0])
    for c in range(NCH):
        nxt = (c + 1) % 2
        if c + 1 < NCH:
            copies[nxt] = pltpu.async_copy(
                table_hbm.at[idx_v.at[pl.ds((c + 1) * CH, CH)]], rows[nxt], sems[nxt]
            )
        copies[c % 2].wait()
        pltpu.sync_copy(rows[c % 2], out_hbm.at[pl.ds(base + c * CH, CH)])


def kernel(indices, table):
    flat_idx = indices.reshape(N).astype(jnp.int32)
    out = _gather_kernel(flat_idx, table)
    word_embeddings = out.reshape(B, L, D)
    lengths = jnp.full((B,), L, dtype=jnp.int32)
    return (word_embeddings, lengths)


# speedup vs baseline: 1.0005x; 1.0000x over previous
import functools

import jax
import jax.numpy as jnp
from jax import lax
from jax.experimental import pallas as pl
from jax.experimental.pallas import tpu as pltpu
from jax.experimental.pallas import tpu_sc as plsc

B = 4096
L = 50
V = 1000000
D = 32

N = B * L
NC = 2
NS = 16
NW = NC * NS
PER_W = N // NW
CH = 800
NCH = PER_W // CH
NBUF = 4

_mesh = plsc.VectorSubcoreMesh(core_axis_name="c", subcore_axis_name="s")


@functools.partial(
    pl.kernel,
    mesh=_mesh,
    compiler_params=pltpu.CompilerParams(use_tc_tiling_on_sc=False),
    out_type=jax.ShapeDtypeStruct((N, D), jnp.float32),
    scratch_types=[
        pltpu.VMEM((PER_W,), jnp.int32),
    ]
    + [pltpu.VMEM((CH, D), jnp.float32) for _ in range(NBUF)]
    + [pltpu.SemaphoreType.DMA for _ in range(2 * NBUF)],
)
def _gather_kernel(idx_hbm, table_hbm, out_hbm, idx_v, *bufs_and_sems):
    rows = bufs_and_sems[:NBUF]
    gsems = bufs_and_sems[NBUF : 2 * NBUF]
    ssems = bufs_and_sems[2 * NBUF :]

    wid = lax.axis_index("s") * NC + lax.axis_index("c")
    base = wid * PER_W

    pltpu.sync_copy(idx_hbm.at[pl.ds(base, PER_W)], idx_v)

    def gather(c, b):
        return pltpu.async_copy(
            table_hbm.at[idx_v.at[pl.ds(c * CH, CH)]], rows[b], gsems[b]
        )

    gath = [gather(i, i) for i in range(NBUF)]
    stor = [None] * NBUF
    for c in range(NCH):
        b = c % NBUF
        gath[b].wait()
        stor[b] = pltpu.async_copy(
            rows[b], out_hbm.at[pl.ds(base + c * CH, CH)], ssems[b]
        )
        if c + NBUF < NCH:
            stor[b].wait()
            gath[b] = gather(c + NBUF, b)
            stor[b] = None
    for s in stor:
        if s is not None:
            s.wait()


def kernel(indices, table):
    flat_idx = indices.reshape(N).astype(jnp.int32)
    out = _gather_kernel(flat_idx, table)
    word_embeddings = out.reshape(B, L, D)
    lengths = jnp.full((B,), L, dtype=jnp.int32)
    return (word_embeddings, lengths)


# speedup vs baseline: 1.2206x; 1.2200x over previous
import functools

import jax
import jax.numpy as jnp
from jax import lax
from jax.experimental import pallas as pl
from jax.experimental.pallas import tpu as pltpu
from jax.experimental.pallas import tpu_sc as plsc

B = 4096
L = 50
V = 1000000
D = 32

N = B * L
NC = 2
NS = 16
NW = NC * NS
PER_W = N // NW
CH = 800
NCH = PER_W // CH
NBUF = 4

_mesh = plsc.VectorSubcoreMesh(core_axis_name="c", subcore_axis_name="s")


@functools.partial(
    pl.kernel,
    mesh=_mesh,
    compiler_params=pltpu.CompilerParams(use_tc_tiling_on_sc=False),
    out_type=jax.ShapeDtypeStruct((B, L, D), jnp.float32),
    scratch_types=[
        pltpu.VMEM((PER_W,), jnp.int32),
    ]
    + [pltpu.VMEM((CH, D), jnp.float32) for _ in range(NBUF)]
    + [pltpu.SemaphoreType.DMA for _ in range(NBUF)],
)
def _gather_kernel(idx_hbm, table_hbm, out_hbm, idx_v, *bufs_and_sems):
    rows = bufs_and_sems[:NBUF]
    gsems = bufs_and_sems[NBUF : 2 * NBUF]

    wid = lax.axis_index("s") * NC + lax.axis_index("c")
    base = wid * PER_W

    pltpu.sync_copy(idx_hbm.at[pl.ds(base, PER_W)], idx_v)

    def gather(c, b):
        return pltpu.async_copy(
            table_hbm.at[idx_v.at[pl.ds(c * CH, CH)]], rows[b], gsems[b]
        )

    gath = [gather(i, i) for i in range(NBUF)]
    BPC = CH // L
    base_b = wid * (PER_W // L)
    for c in range(NCH):
        b = c % NBUF
        gath[b].wait()
        cb0 = base_b + c * BPC
        for i in range(BPC):
            pltpu.sync_copy(
                rows[b].at[pl.ds(i * L, L)], out_hbm.at[cb0 + i]
            )
        if c + NBUF < NCH:
            gath[b] = gather(c + NBUF, b)


def kernel(indices, table):
    flat_idx = indices.reshape(N).astype(jnp.int32)
    word_embeddings = _gather_kernel(flat_idx, table)
    lengths = jnp.full((B,), L, dtype=jnp.int32)
    return (word_embeddings, lengths)
